# baseline (speedup 1.0000x reference)
import functools

import jax
import jax.numpy as jnp
from jax import lax
from jax.experimental import pallas as pl
from jax.experimental.pallas import tpu as pltpu
from jax.experimental.pallas import tpu_sc as plsc

Z = 128
NUM_IN = Z * (Z + 1) // 2
NT = (NUM_IN + Z - 1) // Z
XW = NT * Z
HB = 4


def _softplus16(v):
    t = jnp.exp(-jnp.abs(v))
    u = t / (t + 2.0)
    u2 = u * u
    p = 1.0 + u2 * (
        1.0 / 3.0 + u2 * (1.0 / 5.0 + u2 * (1.0 / 7.0 + u2 * (1.0 / 9.0 + u2 * (1.0 / 11.0))))
    )
    return jnp.maximum(v, 0.0) + 2.0 * u * p


def kernel(x):
    B = x.shape[0]
    info = plsc.get_sparse_core_info()
    NC, NS = info.num_cores, info.num_subcores
    NW = NC * NS
    rows_per_w = B // NW
    n_hb = rows_per_w // HB
    mesh = plsc.VectorSubcoreMesh(core_axis_name="c", subcore_axis_name="s")

    @functools.partial(
        pl.kernel,
        out_type=jax.ShapeDtypeStruct((B * Z * Z,), jnp.float32),
        mesh=mesh,
        compiler_params=pltpu.CompilerParams(needs_layout_passes=False),
        scratch_types=[
            pltpu.VMEM((2 * HB, XW), jnp.float32),
            pltpu.VMEM((2 * Z * Z,), jnp.float32),
            pltpu.SemaphoreType.DMA,
            pltpu.SemaphoreType.DMA,
        ],
    )
    def run(x_hbm, out_hbm, x_v, l_v, in_sem, out_sem):
        wid = lax.axis_index("s") * NC + lax.axis_index("c")
        base = wid * rows_per_w
        zero16 = jnp.zeros((16,), jnp.float32)
        iota16 = lax.iota(jnp.int32, 16)

        @plsc.parallel_loop(0, 2 * Z * Z // 16, unroll=4)
        def _zero(i):
            l_v[pl.ds(i * 16, 16)] = zero16

        def start_in(hb):
            row0 = base + hb * HB
            half = jnp.bitwise_and(hb, 1) * HB

            @plsc.parallel_loop(0, NT, unroll=5)
            def t_body(t):
                col = pl.multiple_of(t * Z, Z)
                pltpu.make_async_copy(
                    x_hbm.at[pl.ds(row0, HB), pl.ds(col, Z)],
                    x_v.at[pl.ds(pl.multiple_of(half, HB), HB), pl.ds(col, Z)],
                    in_sem,
                ).start()

        def wait_in():
            for _ in range(HB):
                pltpu.make_async_copy(
                    out_hbm.at[pl.ds(0, XW)], x_v.at[0], in_sem
                ).wait()

        def out_start(m, lbase):
            pltpu.make_async_copy(
                l_v.at[pl.ds(pl.multiple_of(lbase, Z * Z), Z * Z)],
                out_hbm.at[pl.ds((base + m) * Z * Z, Z * Z)],
                out_sem,
            ).start()

        def out_wait_one():
            pltpu.make_async_copy(
                out_hbm.at[pl.ds(0, Z * Z)], l_v.at[pl.ds(0, Z * Z)], out_sem
            ).wait()

        start_in(0)
        start_in(1)

        def rebuild(row, lbase):
            g_vec = jnp.zeros((16,), jnp.int32) + row
            for j in range(Z // 16 - 1):
                @plsc.parallel_loop(16 * (j + 1), Z, unroll=8)
                def _c(r):
                    off = (r * (r + 1)) // 2
                    l_v[pl.ds(lbase + r * Z + j * 16, 16)] = plsc.load_gather(
                        x_v, [g_vec, off + j * 16 + iota16]
                    )

            @plsc.parallel_loop(0, Z, unroll=4)
            def _b(r):
                k16 = jnp.bitwise_and(r, ~15)
                off = (r * (r + 1)) // 2
                vals = plsc.load_gather(x_v, [g_vec, off + k16 + iota16])
                rr = jnp.bitwise_and(r, 15)
                l_v[pl.ds(lbase + r * Z + k16, 16)] = jnp.where(
                    iota16 < rr, vals, zero16
                )

            @plsc.parallel_loop(0, Z // 16, unroll=2)
            def _d(k8):
                r_vec = iota16 + k8 * 16
                srcv = lax.shift_right_logical(r_vec * (r_vec + 3), 1)
                vals = plsc.load_gather(x_v, [g_vec, srcv])
                sp = _softplus16(vals)
                plsc.store_scatter(l_v, [r_vec * (Z + 1) + lbase], sp)

        def mstep(m, _):
            hb = lax.shift_right_logical(m, 2)
            g = jnp.bitwise_and(m, HB - 1)

            @pl.when(g == 0)
            def _():
                wait_in()

            @pl.when(m >= 2)
            def _():
                out_wait_one()

            row = jnp.bitwise_and(hb, 1) * HB + g
            lbase = jnp.bitwise_and(m, 1) * (Z * Z)
            rebuild(row, lbase)
            out_start(m, lbase)

            @pl.when(jnp.logical_and(g == HB - 1, hb + 2 < n_hb))
            def _():
                start_in(hb + 2)

            return 0

        lax.fori_loop(0, rows_per_w, mstep, 0)

        out_wait_one()
        out_wait_one()

    out = run(x)
    return out.reshape(B, Z, Z)


# speedup vs baseline: 1.0310x; 1.0310x over previous
import functools

import jax
import jax.numpy as jnp
from jax import lax
from jax.experimental import pallas as pl
from jax.experimental.pallas import tpu as pltpu
from jax.experimental.pallas import tpu_sc as plsc

Z = 128
NUM_IN = Z * (Z + 1) // 2
NT = (NUM_IN + Z - 1) // Z
XW = NT * Z
HB = 4


def _softplus16(v):
    t = jnp.exp(-jnp.abs(v))
    u = t / (t + 2.0)
    u2 = u * u
    p = 1.0 + u2 * (
        1.0 / 3.0 + u2 * (1.0 / 5.0 + u2 * (1.0 / 7.0 + u2 * (1.0 / 9.0 + u2 * (1.0 / 11.0))))
    )
    return jnp.maximum(v, 0.0) + 2.0 * u * p


def kernel(x):
    B = x.shape[0]
    info = plsc.get_sparse_core_info()
    NC, NS = info.num_cores, info.num_subcores
    NW = NC * NS
    rows_per_w = B // NW
    n_hb = rows_per_w // HB
    mesh = plsc.VectorSubcoreMesh(core_axis_name="c", subcore_axis_name="s")

    @functools.partial(
        pl.kernel,
        out_type=jax.ShapeDtypeStruct((B * Z * Z,), jnp.float32),
        mesh=mesh,
        compiler_params=pltpu.CompilerParams(needs_layout_passes=False),
        scratch_types=[
            pltpu.VMEM((2 * HB, XW), jnp.float32),
            pltpu.VMEM((2 * Z * Z,), jnp.float32),
            pltpu.SemaphoreType.DMA,
            pltpu.SemaphoreType.DMA,
        ],
    )
    def run(x_hbm, out_hbm, x_v, l_v, in_sem, out_sem):
        wid = lax.axis_index("s") * NC + lax.axis_index("c")
        base = wid * rows_per_w
        zero16 = jnp.zeros((16,), jnp.float32)
        iota16 = lax.iota(jnp.int32, 16)

        @plsc.parallel_loop(0, 2 * Z * Z // 16, unroll=4)
        def _zero(i):
            l_v[pl.ds(i * 16, 16)] = zero16

        def start_in(hb):
            row0 = base + hb * HB
            half = jnp.bitwise_and(hb, 1) * HB

            @plsc.parallel_loop(0, NT, unroll=5)
            def t_body(t):
                col = pl.multiple_of(t * Z, Z)
                pltpu.make_async_copy(
                    x_hbm.at[pl.ds(row0, HB), pl.ds(col, Z)],
                    x_v.at[pl.ds(pl.multiple_of(half, HB), HB), pl.ds(col, Z)],
                    in_sem,
                ).start()

        def wait_in():
            for _ in range(HB):
                pltpu.make_async_copy(
                    out_hbm.at[pl.ds(0, XW)], x_v.at[0], in_sem
                ).wait()

        def out_start(m, lbase):
            pltpu.make_async_copy(
                l_v.at[pl.ds(pl.multiple_of(lbase, Z * Z), Z * Z)],
                out_hbm.at[pl.ds((base + m) * Z * Z, Z * Z)],
                out_sem,
            ).start()

        def out_wait_one():
            pltpu.make_async_copy(
                out_hbm.at[pl.ds(0, Z * Z)], l_v.at[pl.ds(0, Z * Z)], out_sem
            ).wait()

        start_in(0)
        start_in(1)

        def rebuild(row, lbase):
            g_vec = jnp.zeros((16,), jnp.int32) + row
            for j in range(Z // 16 - 1):
                @plsc.parallel_loop(16 * (j + 1), Z, unroll=4)
                def _c(r):
                    off = (r * (r + 1)) // 2
                    l_v[pl.ds(lbase + r * Z + j * 16, 16)] = plsc.load_gather(
                        x_v, [g_vec, off + j * 16 + iota16]
                    )

            @plsc.parallel_loop(0, Z, unroll=4)
            def _b(r):
                k16 = jnp.bitwise_and(r, ~15)
                off = (r * (r + 1)) // 2
                vals = plsc.load_gather(x_v, [g_vec, off + k16 + iota16])
                rr = jnp.bitwise_and(r, 15)
                l_v[pl.ds(lbase + r * Z + k16, 16)] = jnp.where(
                    iota16 < rr, vals, zero16
                )

            @plsc.parallel_loop(0, Z // 16, unroll=2)
            def _d(k8):
                r_vec = iota16 + k8 * 16
                srcv = lax.shift_right_logical(r_vec * (r_vec + 3), 1)
                vals = plsc.load_gather(x_v, [g_vec, srcv])
                sp = _softplus16(vals)
                plsc.store_scatter(l_v, [r_vec * (Z + 1) + lbase], sp)

        def mstep(m, _):
            hb = lax.shift_right_logical(m, 2)
            g = jnp.bitwise_and(m, HB - 1)

            @pl.when(g == 0)
            def _():
                wait_in()

            @pl.when(m >= 2)
            def _():
                out_wait_one()

            row = jnp.bitwise_and(hb, 1) * HB + g
            lbase = jnp.bitwise_and(m, 1) * (Z * Z)
            rebuild(row, lbase)
            out_start(m, lbase)

            @pl.when(jnp.logical_and(g == HB - 1, hb + 2 < n_hb))
            def _():
                start_in(hb + 2)

            return 0

        lax.fori_loop(0, rows_per_w, mstep, 0)

        out_wait_one()
        out_wait_one()

    out = run(x)
    return out.reshape(B, Z, Z)
